# baseline (speedup 1.0000x reference)
import functools

import jax
import jax.numpy as jnp
import numpy as np
from jax import lax
from jax.experimental import pallas as pl
from jax.experimental.pallas import tpu as pltpu
from jax.experimental.pallas import tpu_sc as plsc

N_SC = 8
N_K = 64
N_AG = 128
N_STEP = 91
STEP_FUT = 11
N_JOINT = 32
N_HIST = 11
N_NOSIM = 64

N_FUT = N_STEP - STEP_FUT
ROW = N_STEP * 3
BLK = N_AG * ROW
POS_ROW = N_FUT * 2
YAW_ROW = N_FUT

PI = np.float32(np.pi)
TWO_PI = np.float32(2.0 * np.pi)

_NC = 2
_NS = 16


def _wrap_angle(v):
    m = lax.rem(v, TWO_PI)
    m = jnp.where((m != 0.0) & (m < 0.0), m + TWO_PI, m)
    return m - PI


def _prep_body(coll_ref, rre_ref, role_ref, yaw_ref, cen_ref, nspos_ref,
               nsyaw_ref, idx_ref, trig_ref, posns_ref, yawns_ref):
    col_any = jnp.any(coll_ref[0, :, :, STEP_FUT:], axis=-1)
    rre_any = jnp.any(rre_ref[0, :, :, STEP_FUT:], axis=-1)
    role = jnp.any(role_ref[0], axis=0)
    cnt = col_any.astype(jnp.float32) + rre_any.astype(jnp.float32)
    cnt = cnt * jnp.broadcast_to(role.astype(jnp.float32)[None, :], (N_K, N_AG))
    ones_col = jnp.ones((N_AG, 1), jnp.float32)
    viol_col = lax.dot_general(cnt, ones_col, (((1,), (0,)), ((), ())),
                               precision=lax.Precision.HIGHEST)
    ones_row = jnp.ones((1, N_AG), jnp.float32)
    viol_row = lax.dot_general(ones_row, cnt, (((1,), (1,)), ((), ())),
                               precision=lax.Precision.HIGHEST)

    vi = jnp.broadcast_to(viol_col, (N_K, N_K))
    vj = jnp.broadcast_to(viol_row, (N_K, N_K))
    ii = lax.broadcasted_iota(jnp.int32, (N_K, N_K), 0)
    jj = lax.broadcasted_iota(jnp.int32, (N_K, N_K), 1)
    prec = (vj < vi) | ((vj == vi) & (jj < ii))
    rank_col = jnp.sum(prec.astype(jnp.int32), axis=1, keepdims=True)

    jj2 = lax.broadcasted_iota(jnp.int32, (N_K, N_JOINT), 1)
    ii2 = lax.broadcasted_iota(jnp.int32, (N_K, N_JOINT), 0)
    onehot = jnp.broadcast_to(rank_col, (N_K, N_JOINT)) == jj2
    idx_row = jnp.sum(jnp.where(onehot, ii2, 0), axis=0, keepdims=True)
    idx_ref[...] = idx_row.reshape(1, 1, N_JOINT)

    yaw = yaw_ref[0]
    c = jnp.cos(yaw)
    s = jnp.sin(yaw)
    cx = cen_ref[0, :, 0:1]
    cy = cen_ref[0, :, 1:2]
    rr = lax.broadcasted_iota(jnp.int32, (4, 16), 0)
    hh = lax.broadcasted_iota(jnp.int32, (4, 16), 1) % 2
    cb4 = jnp.broadcast_to(c, (4, 16))
    sb4 = jnp.broadcast_to(s, (4, 16))
    cxb = jnp.broadcast_to(cx, (4, 16))
    cyb = jnp.broadcast_to(cy, (4, 16))
    trig = jnp.where(rr == 0, jnp.where(hh == 0, cb4, sb4),
           jnp.where(rr == 1, jnp.where(hh == 0, -sb4, cb4),
           jnp.where(rr == 2, jnp.where(hh == 0, cxb, cyb),
                     jnp.broadcast_to(yaw, (4, 16)))))
    trig_ref[...] = trig.reshape(1, 4, 16)

    NL = N_HIST * 3
    NQ = N_HIST * 2
    r0 = lax.broadcasted_iota(jnp.int32, (NL, NQ), 0)
    q0 = lax.broadcasted_iota(jnp.int32, (NL, NQ), 1)
    tmatch = (r0 // 3) == (q0 // 2)
    r3 = r0 % 3
    q2 = q0 % 2
    cb = jnp.broadcast_to(c, (NL, NQ))
    sb = jnp.broadcast_to(s, (NL, NQ))
    zero = jnp.zeros((NL, NQ), jnp.float32)
    m = jnp.where(tmatch & (r3 == 0) & (q2 == 0), cb, zero)
    m = m + jnp.where(tmatch & (r3 == 0) & (q2 == 1), sb, zero)
    m = m + jnp.where(tmatch & (r3 == 1) & (q2 == 0), -sb, zero)
    m = m + jnp.where(tmatch & (r3 == 1) & (q2 == 1), cb, zero)
    ns = nspos_ref[0]
    pos = lax.dot_general(ns, m, (((1,), (0,)), ((), ())),
                          precision=lax.Precision.HIGHEST)
    q2r = lax.broadcasted_iota(jnp.int32, (1, NQ), 1) % 2
    bias = jnp.where(q2r == 0, jnp.broadcast_to(cx, (1, NQ)),
                     jnp.broadcast_to(cy, (1, NQ)))
    posns_ref[...] = (pos + bias).reshape(1, N_NOSIM, NQ)

    yv = nsyaw_ref[0] + jnp.broadcast_to(yaw, (N_NOSIM, N_HIST))
    yawns_ref[...] = _wrap_angle(yv + PI).reshape(1, N_NOSIM, N_HIST)


def _prep_call(coll, rre, role_t, yaw3, cen3, nspos, nsyaw):
    grid = (N_SC,)
    return pl.pallas_call(
        _prep_body,
        grid=grid,
        in_specs=[
            pl.BlockSpec((1, N_K, N_AG, N_STEP), lambda s: (s, 0, 0, 0)),
            pl.BlockSpec((1, N_K, N_AG, N_STEP), lambda s: (s, 0, 0, 0)),
            pl.BlockSpec((1, 3, N_AG), lambda s: (s, 0, 0)),
            pl.BlockSpec((1, 1, 1), lambda s: (s, 0, 0)),
            pl.BlockSpec((1, 1, 2), lambda s: (s, 0, 0)),
            pl.BlockSpec((1, N_NOSIM, N_HIST * 3), lambda s: (s, 0, 0)),
            pl.BlockSpec((1, N_NOSIM, N_HIST), lambda s: (s, 0, 0)),
        ],
        out_specs=[
            pl.BlockSpec((1, 1, N_JOINT), lambda s: (s, 0, 0)),
            pl.BlockSpec((1, 4, 16), lambda s: (s, 0, 0)),
            pl.BlockSpec((1, N_NOSIM, N_HIST * 2), lambda s: (s, 0, 0)),
            pl.BlockSpec((1, N_NOSIM, N_HIST), lambda s: (s, 0, 0)),
        ],
        out_shape=[
            jax.ShapeDtypeStruct((N_SC, 1, N_JOINT), jnp.int32),
            jax.ShapeDtypeStruct((N_SC, 4, 16), jnp.float32),
            jax.ShapeDtypeStruct((N_SC, N_NOSIM, N_HIST * 2), jnp.float32),
            jax.ShapeDtypeStruct((N_SC, N_NOSIM, N_HIST), jnp.float32),
        ],
    )(coll, rre, role_t, yaw3, cen3, nspos, nsyaw)


def _sc_body(pred_hbm, idx_hbm, trig_hbm, pos_hbm, yaw_hbm,
             in_v, pos_v, yaw_v, idx_v, trig_v):
    wid = lax.axis_index("s") * _NC + lax.axis_index("c")
    pltpu.sync_copy(idx_hbm, idx_v)
    pltpu.sync_copy(trig_hbm, trig_v)
    lane = lax.broadcasted_iota(jnp.int32, (16,), 0)
    pat_pos = 3 * (lane // 2)
    pat_yaw = 3 * lane

    def task(s, _):
        t = s * N_JOINT + wid
        k_vec = plsc.load_gather(idx_v, [jnp.full((16,), 0, jnp.int32) + t])
        k = k_vec[0]
        pltpu.sync_copy(pred_hbm.at[s, k], in_v)
        base = s * 64
        coef_a = trig_v[pl.ds(base, 16)]
        coef_b = trig_v[pl.ds(base + 16, 16)]
        coef_t = trig_v[pl.ds(base + 32, 16)]
        sy = trig_v[pl.ds(base + 48, 16)]

        def agent(a, _):
            abase = a * ROW + STEP_FUT * 3
            pbase = a * POS_ROW
            ybase = a * YAW_ROW
            for v in range(POS_ROW // 16):
                ix = pat_pos + (abase + 24 * v)
                px = plsc.load_gather(in_v, [ix])
                py = plsc.load_gather(in_v, [ix + 1])
                out = coef_a * px + coef_b * py
                pos_v[pl.ds(pbase + 16 * v, 16)] = out + coef_t
            for w in range(YAW_ROW // 16):
                iy = pat_yaw + (abase + 2 + 48 * w)
                yw = plsc.load_gather(in_v, [iy])
                val = yw + sy
                yaw_v[pl.ds(ybase + 16 * w, 16)] = _wrap_angle(val + PI)
            return 0

        lax.fori_loop(0, N_AG, agent, 0)
        pltpu.sync_copy(pos_v, pos_hbm.at[s, wid])
        pltpu.sync_copy(yaw_v, yaw_hbm.at[s, wid])
        return 0

    lax.fori_loop(0, N_SC, task, 0)


def _sc_call(pred_flat, idx_flat, trig_flat):
    mesh = plsc.VectorSubcoreMesh(core_axis_name="c", subcore_axis_name="s")
    f = pl.kernel(
        _sc_body,
        out_type=(
            jax.ShapeDtypeStruct((N_SC, N_JOINT, N_AG * POS_ROW), jnp.float32),
            jax.ShapeDtypeStruct((N_SC, N_JOINT, N_AG * YAW_ROW), jnp.float32),
        ),
        mesh=mesh,
        compiler_params=pltpu.CompilerParams(needs_layout_passes=False),
        scratch_types=[
            pltpu.VMEM((BLK,), jnp.float32),
            pltpu.VMEM((N_AG * POS_ROW,), jnp.float32),
            pltpu.VMEM((N_AG * YAW_ROW,), jnp.float32),
            pltpu.VMEM((N_SC * N_JOINT,), jnp.int32),
            pltpu.VMEM((N_SC * 64,), jnp.float32),
        ],
    )
    return f(pred_flat, idx_flat, trig_flat)


def kernel(pred_pose, scenario_center, scenario_yaw, agent_pos_hist,
           no_sim_pos, no_sim_yaw, ag_role, collided, run_road_edge,
           valid_sim, valid_no_sim, object_id_sim, object_id_no_sim,
           scenario_id):
    role_t = jnp.swapaxes(ag_role, 1, 2)
    yaw3 = scenario_yaw.reshape(N_SC, 1, 1)
    cen3 = scenario_center.reshape(N_SC, 1, 2)
    nspos = no_sim_pos.reshape(N_SC, N_NOSIM, N_HIST * 3)
    nsyaw = no_sim_yaw.reshape(N_SC, N_NOSIM, N_HIST)

    idx3, trig3, posns, yawns = _prep_call(
        collided, run_road_edge, role_t, yaw3, cen3, nspos, nsyaw)

    pred_flat = pred_pose.reshape(N_SC, N_K, BLK)
    pos_flat, yaw_flat = _sc_call(
        pred_flat, idx3.reshape(N_SC * N_JOINT), trig3.reshape(N_SC * 64))

    pos_sim = pos_flat.reshape(N_SC, N_JOINT, N_AG, N_FUT, 2)
    yaw_sim = yaw_flat.reshape(N_SC, N_JOINT, N_AG, N_FUT, 1)
    pos_ns = posns.reshape(N_SC, N_NOSIM, N_HIST, 2)
    yaw_ns = yawns.reshape(N_SC, N_NOSIM, N_HIST, 1)
    z_sim = agent_pos_hist[..., 2:3]
    z_ns = no_sim_pos[..., 2:3]
    return (scenario_id, valid_sim, pos_sim, z_sim, yaw_sim, valid_no_sim,
            object_id_sim, pos_ns, z_ns, yaw_ns, object_id_no_sim)


# speedup vs baseline: 3.4790x; 3.4790x over previous
import jax
import jax.numpy as jnp
import numpy as np
from jax import lax
from jax.experimental import pallas as pl
from jax.experimental.pallas import tpu as pltpu
from jax.experimental.pallas import tpu_sc as plsc

N_SC = 8
N_K = 64
N_AG = 128
N_STEP = 91
STEP_FUT = 11
N_JOINT = 32
N_HIST = 11
N_NOSIM = 64

N_FUT = N_STEP - STEP_FUT
ROW3 = N_STEP * 3

PI = np.float32(np.pi)
TWO_PI = np.float32(2.0 * np.pi)

_NC = 2
_NS = 16


def _wrap_angle(v):
    m = lax.rem(v, TWO_PI)
    m = jnp.where((m != 0.0) & (m < 0.0), m + TWO_PI, m)
    return m - PI


def _prep_body(coll_ref, rre_ref, role_ref, yaw_ref, cen_ref, nspos_ref,
               nsyaw_ref, idx_ref, trig_ref, posns_ref, yawns_ref):
    col_any = jnp.max(coll_ref[0, STEP_FUT:, :, :].astype(jnp.float32), axis=0)
    rre_any = jnp.max(rre_ref[0, STEP_FUT:, :, :].astype(jnp.float32), axis=0)
    role = jnp.max(role_ref[0].astype(jnp.float32), axis=0, keepdims=True)
    cnt = (col_any + rre_any) * jnp.broadcast_to(role, (N_K, N_AG))
    ones_col = jnp.ones((N_AG, 1), jnp.float32)
    viol_col = lax.dot_general(cnt, ones_col, (((1,), (0,)), ((), ())),
                               precision=lax.Precision.HIGHEST)
    ones_row = jnp.ones((1, N_AG), jnp.float32)
    viol_row = lax.dot_general(ones_row, cnt, (((1,), (1,)), ((), ())),
                               precision=lax.Precision.HIGHEST)

    vi = jnp.broadcast_to(viol_col, (N_K, N_K))
    vj = jnp.broadcast_to(viol_row, (N_K, N_K))
    ii = lax.broadcasted_iota(jnp.int32, (N_K, N_K), 0)
    jj = lax.broadcasted_iota(jnp.int32, (N_K, N_K), 1)
    prec = (vj < vi) | ((vj == vi) & (jj < ii))
    rank_col = jnp.sum(prec.astype(jnp.int32), axis=1, keepdims=True)

    jj2 = lax.broadcasted_iota(jnp.int32, (N_K, N_JOINT), 1)
    ii2 = lax.broadcasted_iota(jnp.int32, (N_K, N_JOINT), 0)
    onehot = jnp.broadcast_to(rank_col, (N_K, N_JOINT)) == jj2
    idx_row = jnp.sum(jnp.where(onehot, ii2, 0), axis=0, keepdims=True)
    idx_ref[...] = idx_row.reshape(1, 1, N_JOINT)

    yaw = yaw_ref[0]
    c = jnp.cos(yaw)
    s = jnp.sin(yaw)
    cx = cen_ref[0, :, 0:1]
    cy = cen_ref[0, :, 1:2]
    rr8 = lax.broadcasted_iota(jnp.int32, (8, 16), 0)
    trig = jnp.where(rr8 == 0, jnp.broadcast_to(c, (8, 16)),
           jnp.where(rr8 == 1, jnp.broadcast_to(s, (8, 16)),
           jnp.where(rr8 == 2, jnp.broadcast_to(cx, (8, 16)),
           jnp.where(rr8 == 3, jnp.broadcast_to(cy, (8, 16)),
                     jnp.broadcast_to(yaw, (8, 16))))))
    trig_ref[...] = trig.reshape(1, 8, 16)

    NL = N_HIST * 3
    NQ = N_HIST * 2
    r0 = lax.broadcasted_iota(jnp.int32, (NL, NQ), 0)
    q0 = lax.broadcasted_iota(jnp.int32, (NL, NQ), 1)
    tmatch = (r0 // 3) == (q0 // 2)
    r3 = r0 % 3
    q2 = q0 % 2
    cb = jnp.broadcast_to(c, (NL, NQ))
    sb = jnp.broadcast_to(s, (NL, NQ))
    zero = jnp.zeros((NL, NQ), jnp.float32)
    m = jnp.where(tmatch & (r3 == 0) & (q2 == 0), cb, zero)
    m = m + jnp.where(tmatch & (r3 == 0) & (q2 == 1), sb, zero)
    m = m + jnp.where(tmatch & (r3 == 1) & (q2 == 0), -sb, zero)
    m = m + jnp.where(tmatch & (r3 == 1) & (q2 == 1), cb, zero)
    ns = nspos_ref[0]
    pos = lax.dot_general(ns, m, (((1,), (0,)), ((), ())),
                          precision=lax.Precision.HIGHEST)
    q2r = lax.broadcasted_iota(jnp.int32, (1, NQ), 1) % 2
    bias = jnp.where(q2r == 0, jnp.broadcast_to(cx, (1, NQ)),
                     jnp.broadcast_to(cy, (1, NQ)))
    posns_ref[...] = (pos + bias).reshape(1, N_NOSIM, NQ)

    yv = nsyaw_ref[0] + jnp.broadcast_to(yaw, (N_NOSIM, N_HIST))
    yawns_ref[...] = _wrap_angle(yv + PI).reshape(1, N_NOSIM, N_HIST)


def _prep_call(coll, rre, role_t, yaw3, cen3, nspos, nsyaw):
    return pl.pallas_call(
        _prep_body,
        grid=(N_SC,),
        in_specs=[
            pl.BlockSpec((1, N_STEP, N_K, N_AG), lambda s: (s, 0, 0, 0)),
            pl.BlockSpec((1, N_STEP, N_K, N_AG), lambda s: (s, 0, 0, 0)),
            pl.BlockSpec((1, 3, N_AG), lambda s: (s, 0, 0)),
            pl.BlockSpec((1, 1, 1), lambda s: (s, 0, 0)),
            pl.BlockSpec((1, 1, 2), lambda s: (s, 0, 0)),
            pl.BlockSpec((1, N_NOSIM, N_HIST * 3), lambda s: (s, 0, 0)),
            pl.BlockSpec((1, N_NOSIM, N_HIST), lambda s: (s, 0, 0)),
        ],
        out_specs=[
            pl.BlockSpec((1, 1, N_JOINT), lambda s: (s, 0, 0)),
            pl.BlockSpec((1, 8, 16), lambda s: (s, 0, 0)),
            pl.BlockSpec((1, N_NOSIM, N_HIST * 2), lambda s: (s, 0, 0)),
            pl.BlockSpec((1, N_NOSIM, N_HIST), lambda s: (s, 0, 0)),
        ],
        out_shape=[
            jax.ShapeDtypeStruct((N_SC, 1, N_JOINT), jnp.int32),
            jax.ShapeDtypeStruct((N_SC, 8, 16), jnp.float32),
            jax.ShapeDtypeStruct((N_SC, N_NOSIM, N_HIST * 2), jnp.float32),
            jax.ShapeDtypeStruct((N_SC, N_NOSIM, N_HIST), jnp.float32),
        ],
    )(coll, rre, role_t, yaw3, cen3, nspos, nsyaw)


def _sc_body(pred_hbm, idx_hbm, trig_hbm, pos_hbm, yaw_hbm,
             in_v, pos_v, yaw_v, idx_v, trig_v):
    wid = lax.axis_index("s") * _NC + lax.axis_index("c")
    pltpu.sync_copy(idx_hbm, idx_v)
    pltpu.sync_copy(trig_hbm, trig_v)

    def task(s, _):
        t = s * N_JOINT + wid
        k_vec = plsc.load_gather(idx_v, [jnp.full((16,), 0, jnp.int32) + t])
        k = k_vec[0]
        pltpu.sync_copy(pred_hbm.at[s, :, k], in_v)
        base = s * 128
        c = trig_v[pl.ds(base, 16)]
        sn = trig_v[pl.ds(base + 16, 16)]
        cx = trig_v[pl.ds(base + 32, 16)]
        cy = trig_v[pl.ds(base + 48, 16)]
        sy = trig_v[pl.ds(base + 64, 16)]

        def step(tt, _):
            r = STEP_FUT * 3 + tt * 3
            for g in range(N_AG // 16):
                sl = pl.ds(16 * g, 16)
                x = in_v[r, sl]
                y = in_v[r + 1, sl]
                w = in_v[r + 2, sl]
                pos_v[2 * tt, sl] = (c * x - sn * y) + cx
                pos_v[2 * tt + 1, sl] = (sn * x + c * y) + cy
                val = w + sy
                yaw_v[tt, sl] = _wrap_angle(val + PI)
            return 0

        lax.fori_loop(0, N_FUT, step, 0)
        pltpu.sync_copy(pos_v, pos_hbm.at[s, wid])
        pltpu.sync_copy(yaw_v, yaw_hbm.at[s, wid])
        return 0

    lax.fori_loop(0, N_SC, task, 0)


def _sc_call(pred_r, idx_flat, trig_flat):
    mesh = plsc.VectorSubcoreMesh(core_axis_name="c", subcore_axis_name="s")
    f = pl.kernel(
        _sc_body,
        out_type=(
            jax.ShapeDtypeStruct((N_SC, N_JOINT, N_FUT * 2, N_AG), jnp.float32),
            jax.ShapeDtypeStruct((N_SC, N_JOINT, N_FUT, N_AG), jnp.float32),
        ),
        mesh=mesh,
        compiler_params=pltpu.CompilerParams(needs_layout_passes=False),
        scratch_types=[
            pltpu.VMEM((ROW3, N_AG), jnp.float32),
            pltpu.VMEM((N_FUT * 2, N_AG), jnp.float32),
            pltpu.VMEM((N_FUT, N_AG), jnp.float32),
            pltpu.VMEM((N_SC * N_JOINT,), jnp.int32),
            pltpu.VMEM((N_SC * 128,), jnp.float32),
        ],
    )
    return f(pred_r, idx_flat, trig_flat)


def kernel(pred_pose, scenario_center, scenario_yaw, agent_pos_hist,
           no_sim_pos, no_sim_yaw, ag_role, collided, run_road_edge,
           valid_sim, valid_no_sim, object_id_sim, object_id_no_sim,
           scenario_id):
    coll8 = jnp.transpose(collided, (0, 3, 1, 2)).astype(jnp.int8)
    rre8 = jnp.transpose(run_road_edge, (0, 3, 1, 2)).astype(jnp.int8)
    role8 = jnp.swapaxes(ag_role, 1, 2).astype(jnp.int8)
    yaw3 = scenario_yaw.reshape(N_SC, 1, 1)
    cen3 = scenario_center.reshape(N_SC, 1, 2)
    nspos = no_sim_pos.reshape(N_SC, N_NOSIM, N_HIST * 3)
    nsyaw = no_sim_yaw.reshape(N_SC, N_NOSIM, N_HIST)

    idx3, trig3, posns, yawns = _prep_call(
        coll8, rre8, role8, yaw3, cen3, nspos, nsyaw)

    pred_r = jnp.transpose(pred_pose, (0, 3, 4, 1, 2)).reshape(
        N_SC, ROW3, N_K, N_AG)
    pos_flat, yaw_flat = _sc_call(
        pred_r, idx3.reshape(N_SC * N_JOINT), trig3.reshape(N_SC * 128))

    pos_sim = jnp.transpose(
        pos_flat.reshape(N_SC, N_JOINT, N_FUT, 2, N_AG), (0, 1, 4, 2, 3))
    yaw_sim = jnp.transpose(
        yaw_flat.reshape(N_SC, N_JOINT, N_FUT, 1, N_AG), (0, 1, 4, 2, 3))
    pos_ns = posns.reshape(N_SC, N_NOSIM, N_HIST, 2)
    yaw_ns = yawns.reshape(N_SC, N_NOSIM, N_HIST, 1)
    z_sim = agent_pos_hist[..., 2:3]
    z_ns = no_sim_pos[..., 2:3]
    return (scenario_id, valid_sim, pos_sim, z_sim, yaw_sim, valid_no_sim,
            object_id_sim, pos_ns, z_ns, yaw_ns, object_id_no_sim)


# speedup vs baseline: 3.7906x; 1.0896x over previous
import jax
import jax.numpy as jnp
import numpy as np
from jax import lax
from jax.experimental import pallas as pl
from jax.experimental.pallas import tpu as pltpu
from jax.experimental.pallas import tpu_sc as plsc

N_SC = 8
N_K = 64
N_AG = 128
N_STEP = 91
STEP_FUT = 11
N_JOINT = 32
N_HIST = 11
N_NOSIM = 64

N_FUT = N_STEP - STEP_FUT
ROW3 = N_STEP * 3

PI = np.float32(np.pi)
TWO_PI = np.float32(2.0 * np.pi)

_NC = 2
_NS = 16


def _wrap_angle(v):
    m = lax.rem(v, TWO_PI)
    m = jnp.where((m != 0.0) & (m < 0.0), m + TWO_PI, m)
    return m - PI


def _prep_body(coll_ref, rre_ref, role_ref, yaw_ref, cen_ref, nspos_ref,
               nsyaw_ref, idx_ref, trig_ref, posns_ref, yawns_ref):
    col_any = jnp.max(coll_ref[0, STEP_FUT:, :, :].astype(jnp.float32), axis=0)
    rre_any = jnp.max(rre_ref[0, STEP_FUT:, :, :].astype(jnp.float32), axis=0)
    role = jnp.max(role_ref[0].astype(jnp.float32), axis=0, keepdims=True)
    cnt = (col_any + rre_any) * jnp.broadcast_to(role, (N_K, N_AG))
    ones_col = jnp.ones((N_AG, 1), jnp.float32)
    viol_col = lax.dot_general(cnt, ones_col, (((1,), (0,)), ((), ())),
                               precision=lax.Precision.HIGHEST)
    ones_row = jnp.ones((1, N_AG), jnp.float32)
    viol_row = lax.dot_general(ones_row, cnt, (((1,), (1,)), ((), ())),
                               precision=lax.Precision.HIGHEST)

    vi = jnp.broadcast_to(viol_col, (N_K, N_K))
    vj = jnp.broadcast_to(viol_row, (N_K, N_K))
    ii = lax.broadcasted_iota(jnp.int32, (N_K, N_K), 0)
    jj = lax.broadcasted_iota(jnp.int32, (N_K, N_K), 1)
    prec = (vj < vi) | ((vj == vi) & (jj < ii))
    rank_col = jnp.sum(prec.astype(jnp.int32), axis=1, keepdims=True)

    jj2 = lax.broadcasted_iota(jnp.int32, (N_K, N_JOINT), 1)
    ii2 = lax.broadcasted_iota(jnp.int32, (N_K, N_JOINT), 0)
    onehot = jnp.broadcast_to(rank_col, (N_K, N_JOINT)) == jj2
    idx_row = jnp.sum(jnp.where(onehot, ii2, 0), axis=0, keepdims=True)
    idx_ref[...] = idx_row.reshape(1, 1, N_JOINT)

    yaw = yaw_ref[0]
    c = jnp.cos(yaw)
    s = jnp.sin(yaw)
    cx = cen_ref[0, :, 0:1]
    cy = cen_ref[0, :, 1:2]
    rr8 = lax.broadcasted_iota(jnp.int32, (8, 16), 0)
    trig = jnp.where(rr8 == 0, jnp.broadcast_to(c, (8, 16)),
           jnp.where(rr8 == 1, jnp.broadcast_to(s, (8, 16)),
           jnp.where(rr8 == 2, jnp.broadcast_to(cx, (8, 16)),
           jnp.where(rr8 == 3, jnp.broadcast_to(cy, (8, 16)),
                     jnp.broadcast_to(yaw, (8, 16))))))
    trig_ref[...] = trig.reshape(1, 8, 16)

    NL = N_HIST * 3
    NQ = N_HIST * 2
    r0 = lax.broadcasted_iota(jnp.int32, (NL, NQ), 0)
    q0 = lax.broadcasted_iota(jnp.int32, (NL, NQ), 1)
    tmatch = (r0 // 3) == (q0 // 2)
    r3 = r0 % 3
    q2 = q0 % 2
    cb = jnp.broadcast_to(c, (NL, NQ))
    sb = jnp.broadcast_to(s, (NL, NQ))
    zero = jnp.zeros((NL, NQ), jnp.float32)
    m = jnp.where(tmatch & (r3 == 0) & (q2 == 0), cb, zero)
    m = m + jnp.where(tmatch & (r3 == 0) & (q2 == 1), sb, zero)
    m = m + jnp.where(tmatch & (r3 == 1) & (q2 == 0), -sb, zero)
    m = m + jnp.where(tmatch & (r3 == 1) & (q2 == 1), cb, zero)
    ns = nspos_ref[0]
    pos = lax.dot_general(ns, m, (((1,), (0,)), ((), ())),
                          precision=lax.Precision.HIGHEST)
    q2r = lax.broadcasted_iota(jnp.int32, (1, NQ), 1) % 2
    bias = jnp.where(q2r == 0, jnp.broadcast_to(cx, (1, NQ)),
                     jnp.broadcast_to(cy, (1, NQ)))
    posns_ref[...] = (pos + bias).reshape(1, N_NOSIM, NQ)

    yv = nsyaw_ref[0] + jnp.broadcast_to(yaw, (N_NOSIM, N_HIST))
    yawns_ref[...] = _wrap_angle(yv + PI).reshape(1, N_NOSIM, N_HIST)


def _prep_call(coll, rre, role_t, yaw3, cen3, nspos, nsyaw):
    return pl.pallas_call(
        _prep_body,
        grid=(N_SC,),
        in_specs=[
            pl.BlockSpec((1, N_STEP, N_K, N_AG), lambda s: (s, 0, 0, 0)),
            pl.BlockSpec((1, N_STEP, N_K, N_AG), lambda s: (s, 0, 0, 0)),
            pl.BlockSpec((1, 3, N_AG), lambda s: (s, 0, 0)),
            pl.BlockSpec((1, 1, 1), lambda s: (s, 0, 0)),
            pl.BlockSpec((1, 1, 2), lambda s: (s, 0, 0)),
            pl.BlockSpec((1, N_NOSIM, N_HIST * 3), lambda s: (s, 0, 0)),
            pl.BlockSpec((1, N_NOSIM, N_HIST), lambda s: (s, 0, 0)),
        ],
        out_specs=[
            pl.BlockSpec((1, 1, N_JOINT), lambda s: (s, 0, 0)),
            pl.BlockSpec((1, 8, 16), lambda s: (s, 0, 0)),
            pl.BlockSpec((1, N_NOSIM, N_HIST * 2), lambda s: (s, 0, 0)),
            pl.BlockSpec((1, N_NOSIM, N_HIST), lambda s: (s, 0, 0)),
        ],
        out_shape=[
            jax.ShapeDtypeStruct((N_SC, 1, N_JOINT), jnp.int32),
            jax.ShapeDtypeStruct((N_SC, 8, 16), jnp.float32),
            jax.ShapeDtypeStruct((N_SC, N_NOSIM, N_HIST * 2), jnp.float32),
            jax.ShapeDtypeStruct((N_SC, N_NOSIM, N_HIST), jnp.float32),
        ],
    )(coll, rre, role_t, yaw3, cen3, nspos, nsyaw)


def _sc_body(pred_hbm, idx_hbm, trig_hbm, pos_hbm, yaw_hbm,
             in_v0, in_v1, pos_v, yaw_v, idx_v, trig_v,
             sem_in0, sem_in1, sem_pos, sem_yaw):
    wid = lax.axis_index("s") * _NC + lax.axis_index("c")
    pltpu.sync_copy(idx_hbm, idx_v)
    pltpu.sync_copy(trig_hbm, trig_v)
    bufs = (in_v0, in_v1)
    sems = (sem_in0, sem_in1)

    def start_in(s):
        t = s * N_JOINT + wid
        k = plsc.load_gather(idx_v, [jnp.full((16,), 0, jnp.int32) + t])[0]
        return pltpu.async_copy(
            pred_hbm.at[s, pl.ds(STEP_FUT * 3, N_FUT * 3), k],
            bufs[s % 2], sems[s % 2])

    cp_in = start_in(0)
    cp_pos = cp_yaw = None
    for s in range(N_SC):
        buf = bufs[s % 2]
        cp_in.wait()
        if s + 1 < N_SC:
            cp_in = start_in(s + 1)
        base = s * 128
        c = trig_v[pl.ds(base, 16)]
        sn = trig_v[pl.ds(base + 16, 16)]
        cx = trig_v[pl.ds(base + 32, 16)]
        cy = trig_v[pl.ds(base + 48, 16)]
        sy = trig_v[pl.ds(base + 64, 16)]
        if cp_pos is not None:
            cp_pos.wait()
            cp_yaw.wait()

        def step(tt, _, buf=buf, c=c, sn=sn, cx=cx, cy=cy, sy=sy):
            r = tt * 3
            for g in range(N_AG // 16):
                sl = pl.ds(16 * g, 16)
                x = buf[r, sl]
                y = buf[r + 1, sl]
                w = buf[r + 2, sl]
                pos_v[2 * tt, sl] = (c * x - sn * y) + cx
                pos_v[2 * tt + 1, sl] = (sn * x + c * y) + cy
                val = w + sy
                yaw_v[tt, sl] = _wrap_angle(val + PI)
            return 0

        lax.fori_loop(0, N_FUT, step, 0)
        cp_pos = pltpu.async_copy(pos_v, pos_hbm.at[s, wid], sem_pos)
        cp_yaw = pltpu.async_copy(yaw_v, yaw_hbm.at[s, wid], sem_yaw)
    cp_pos.wait()
    cp_yaw.wait()


def _sc_call(pred_r, idx_flat, trig_flat):
    mesh = plsc.VectorSubcoreMesh(core_axis_name="c", subcore_axis_name="s")
    f = pl.kernel(
        _sc_body,
        out_type=(
            jax.ShapeDtypeStruct((N_SC, N_JOINT, N_FUT * 2, N_AG), jnp.float32),
            jax.ShapeDtypeStruct((N_SC, N_JOINT, N_FUT, N_AG), jnp.float32),
        ),
        mesh=mesh,
        compiler_params=pltpu.CompilerParams(needs_layout_passes=False),
        scratch_types=[
            pltpu.VMEM((N_FUT * 3, N_AG), jnp.float32),
            pltpu.VMEM((N_FUT * 3, N_AG), jnp.float32),
            pltpu.VMEM((N_FUT * 2, N_AG), jnp.float32),
            pltpu.VMEM((N_FUT, N_AG), jnp.float32),
            pltpu.VMEM((N_SC * N_JOINT,), jnp.int32),
            pltpu.VMEM((N_SC * 128,), jnp.float32),
            pltpu.SemaphoreType.DMA,
            pltpu.SemaphoreType.DMA,
            pltpu.SemaphoreType.DMA,
            pltpu.SemaphoreType.DMA,
        ],
    )
    return f(pred_r, idx_flat, trig_flat)


def kernel(pred_pose, scenario_center, scenario_yaw, agent_pos_hist,
           no_sim_pos, no_sim_yaw, ag_role, collided, run_road_edge,
           valid_sim, valid_no_sim, object_id_sim, object_id_no_sim,
           scenario_id):
    coll8 = jnp.transpose(collided, (0, 3, 1, 2)).astype(jnp.int8)
    rre8 = jnp.transpose(run_road_edge, (0, 3, 1, 2)).astype(jnp.int8)
    role8 = jnp.swapaxes(ag_role, 1, 2).astype(jnp.int8)
    yaw3 = scenario_yaw.reshape(N_SC, 1, 1)
    cen3 = scenario_center.reshape(N_SC, 1, 2)
    nspos = no_sim_pos.reshape(N_SC, N_NOSIM, N_HIST * 3)
    nsyaw = no_sim_yaw.reshape(N_SC, N_NOSIM, N_HIST)

    idx3, trig3, posns, yawns = _prep_call(
        coll8, rre8, role8, yaw3, cen3, nspos, nsyaw)

    pred_r = jnp.transpose(pred_pose, (0, 3, 4, 1, 2)).reshape(
        N_SC, ROW3, N_K, N_AG)
    pos_flat, yaw_flat = _sc_call(
        pred_r, idx3.reshape(N_SC * N_JOINT), trig3.reshape(N_SC * 128))

    pos_sim = jnp.transpose(
        pos_flat.reshape(N_SC, N_JOINT, N_FUT, 2, N_AG), (0, 1, 4, 2, 3))
    yaw_sim = jnp.transpose(
        yaw_flat.reshape(N_SC, N_JOINT, N_FUT, 1, N_AG), (0, 1, 4, 2, 3))
    pos_ns = posns.reshape(N_SC, N_NOSIM, N_HIST, 2)
    yaw_ns = yawns.reshape(N_SC, N_NOSIM, N_HIST, 1)
    z_sim = agent_pos_hist[..., 2:3]
    z_ns = no_sim_pos[..., 2:3]
    return (scenario_id, valid_sim, pos_sim, z_sim, yaw_sim, valid_no_sim,
            object_id_sim, pos_ns, z_ns, yaw_ns, object_id_no_sim)


# speedup vs baseline: 3.8587x; 1.0180x over previous
import jax
import jax.numpy as jnp
import numpy as np
from jax import lax
from jax.experimental import pallas as pl
from jax.experimental.pallas import tpu as pltpu
from jax.experimental.pallas import tpu_sc as plsc

N_SC = 8
N_K = 64
N_AG = 128
N_STEP = 91
STEP_FUT = 11
N_JOINT = 32
N_HIST = 11
N_NOSIM = 64

N_FUT = N_STEP - STEP_FUT
ROW3 = N_STEP * 3

PI = np.float32(np.pi)
TWO_PI = np.float32(2.0 * np.pi)
INV_2PI = np.float32(1.0 / (2.0 * np.pi))

_NC = 2
_NS = 16


def _wrap_angle(v):
    m = lax.rem(v, TWO_PI)
    m = jnp.where((m != 0.0) & (m < 0.0), m + TWO_PI, m)
    return m - PI


def _wrap_angle_fast(v):
    q = (v * INV_2PI).astype(jnp.int32).astype(jnp.float32)
    m = v - q * TWO_PI
    m = jnp.where(m < 0.0, m + TWO_PI, m)
    return m - PI


def _prep_body(coll_ref, rre_ref, role_ref, yaw_ref, cen_ref, nspos_ref,
               nsyaw_ref, idx_ref, trig_ref, posns_ref, yawns_ref):
    col_any = jnp.max(coll_ref[0, STEP_FUT:, :, :].astype(jnp.float32), axis=0)
    rre_any = jnp.max(rre_ref[0, STEP_FUT:, :, :].astype(jnp.float32), axis=0)
    role = jnp.max(role_ref[0].astype(jnp.float32), axis=0, keepdims=True)
    cnt = (col_any + rre_any) * jnp.broadcast_to(role, (N_K, N_AG))
    ones_col = jnp.ones((N_AG, 1), jnp.float32)
    viol_col = lax.dot_general(cnt, ones_col, (((1,), (0,)), ((), ())),
                               precision=lax.Precision.HIGHEST)
    ones_row = jnp.ones((1, N_AG), jnp.float32)
    viol_row = lax.dot_general(ones_row, cnt, (((1,), (1,)), ((), ())),
                               precision=lax.Precision.HIGHEST)

    vi = jnp.broadcast_to(viol_col, (N_K, N_K))
    vj = jnp.broadcast_to(viol_row, (N_K, N_K))
    ii = lax.broadcasted_iota(jnp.int32, (N_K, N_K), 0)
    jj = lax.broadcasted_iota(jnp.int32, (N_K, N_K), 1)
    prec = (vj < vi) | ((vj == vi) & (jj < ii))
    rank_col = jnp.sum(prec.astype(jnp.int32), axis=1, keepdims=True)

    jj2 = lax.broadcasted_iota(jnp.int32, (N_K, N_JOINT), 1)
    ii2 = lax.broadcasted_iota(jnp.int32, (N_K, N_JOINT), 0)
    onehot = jnp.broadcast_to(rank_col, (N_K, N_JOINT)) == jj2
    idx_row = jnp.sum(jnp.where(onehot, ii2, 0), axis=0, keepdims=True)
    idx_ref[...] = idx_row.reshape(1, 1, N_JOINT)

    yaw = yaw_ref[0]
    c = jnp.cos(yaw)
    s = jnp.sin(yaw)
    cx = cen_ref[0, :, 0:1]
    cy = cen_ref[0, :, 1:2]
    rr8 = lax.broadcasted_iota(jnp.int32, (8, 16), 0)
    trig = jnp.where(rr8 == 0, jnp.broadcast_to(c, (8, 16)),
           jnp.where(rr8 == 1, jnp.broadcast_to(s, (8, 16)),
           jnp.where(rr8 == 2, jnp.broadcast_to(cx, (8, 16)),
           jnp.where(rr8 == 3, jnp.broadcast_to(cy, (8, 16)),
                     jnp.broadcast_to(yaw, (8, 16))))))
    trig_ref[...] = trig.reshape(1, 8, 16)

    NL = N_HIST * 3
    NQ = N_HIST * 2
    r0 = lax.broadcasted_iota(jnp.int32, (NL, NQ), 0)
    q0 = lax.broadcasted_iota(jnp.int32, (NL, NQ), 1)
    tmatch = (r0 // 3) == (q0 // 2)
    r3 = r0 % 3
    q2 = q0 % 2
    cb = jnp.broadcast_to(c, (NL, NQ))
    sb = jnp.broadcast_to(s, (NL, NQ))
    zero = jnp.zeros((NL, NQ), jnp.float32)
    m = jnp.where(tmatch & (r3 == 0) & (q2 == 0), cb, zero)
    m = m + jnp.where(tmatch & (r3 == 0) & (q2 == 1), sb, zero)
    m = m + jnp.where(tmatch & (r3 == 1) & (q2 == 0), -sb, zero)
    m = m + jnp.where(tmatch & (r3 == 1) & (q2 == 1), cb, zero)
    ns = nspos_ref[0]
    pos = lax.dot_general(ns, m, (((1,), (0,)), ((), ())),
                          precision=lax.Precision.HIGHEST)
    q2r = lax.broadcasted_iota(jnp.int32, (1, NQ), 1) % 2
    bias = jnp.where(q2r == 0, jnp.broadcast_to(cx, (1, NQ)),
                     jnp.broadcast_to(cy, (1, NQ)))
    posns_ref[...] = (pos + bias).reshape(1, N_NOSIM, NQ)

    yv = nsyaw_ref[0] + jnp.broadcast_to(yaw, (N_NOSIM, N_HIST))
    yawns_ref[...] = _wrap_angle(yv + PI).reshape(1, N_NOSIM, N_HIST)


def _prep_call(coll, rre, role_t, yaw3, cen3, nspos, nsyaw):
    return pl.pallas_call(
        _prep_body,
        grid=(N_SC,),
        in_specs=[
            pl.BlockSpec((1, N_STEP, N_K, N_AG), lambda s: (s, 0, 0, 0)),
            pl.BlockSpec((1, N_STEP, N_K, N_AG), lambda s: (s, 0, 0, 0)),
            pl.BlockSpec((1, 3, N_AG), lambda s: (s, 0, 0)),
            pl.BlockSpec((1, 1, 1), lambda s: (s, 0, 0)),
            pl.BlockSpec((1, 1, 2), lambda s: (s, 0, 0)),
            pl.BlockSpec((1, N_NOSIM, N_HIST * 3), lambda s: (s, 0, 0)),
            pl.BlockSpec((1, N_NOSIM, N_HIST), lambda s: (s, 0, 0)),
        ],
        out_specs=[
            pl.BlockSpec((1, 1, N_JOINT), lambda s: (s, 0, 0)),
            pl.BlockSpec((1, 8, 16), lambda s: (s, 0, 0)),
            pl.BlockSpec((1, N_NOSIM, N_HIST * 2), lambda s: (s, 0, 0)),
            pl.BlockSpec((1, N_NOSIM, N_HIST), lambda s: (s, 0, 0)),
        ],
        out_shape=[
            jax.ShapeDtypeStruct((N_SC, 1, N_JOINT), jnp.int32),
            jax.ShapeDtypeStruct((N_SC, 8, 16), jnp.float32),
            jax.ShapeDtypeStruct((N_SC, N_NOSIM, N_HIST * 2), jnp.float32),
            jax.ShapeDtypeStruct((N_SC, N_NOSIM, N_HIST), jnp.float32),
        ],
    )(coll, rre, role_t, yaw3, cen3, nspos, nsyaw)


def _sc_body(pred_hbm, idx_hbm, trig_hbm, pos_hbm, yaw_hbm,
             in_v0, in_v1, pos_v, yaw_v, idx_v, trig_v,
             sem_in0, sem_in1, sem_pos, sem_yaw):
    wid = lax.axis_index("s") * _NC + lax.axis_index("c")
    pltpu.sync_copy(idx_hbm, idx_v)
    pltpu.sync_copy(trig_hbm, trig_v)
    bufs = (in_v0, in_v1)
    sems = (sem_in0, sem_in1)

    def start_in(s):
        t = s * N_JOINT + wid
        k = plsc.load_gather(idx_v, [jnp.full((16,), 0, jnp.int32) + t])[0]
        return pltpu.async_copy(
            pred_hbm.at[s, pl.ds(STEP_FUT * 3, N_FUT * 3), k],
            bufs[s % 2], sems[s % 2])

    cp_in = start_in(0)
    cp_pos = cp_yaw = None
    for s in range(N_SC):
        buf = bufs[s % 2]
        cp_in.wait()
        if s + 1 < N_SC:
            cp_in = start_in(s + 1)
        base = s * 128
        c = trig_v[pl.ds(base, 16)]
        sn = trig_v[pl.ds(base + 16, 16)]
        cx = trig_v[pl.ds(base + 32, 16)]
        cy = trig_v[pl.ds(base + 48, 16)]
        sy = trig_v[pl.ds(base + 64, 16)]
        if cp_pos is not None:
            cp_pos.wait()
            cp_yaw.wait()

        def step(tt, _, buf=buf, c=c, sn=sn, cx=cx, cy=cy, sy=sy):
            r = tt * 3
            for g in range(N_AG // 16):
                sl = pl.ds(16 * g, 16)
                x = buf[r, sl]
                y = buf[r + 1, sl]
                w = buf[r + 2, sl]
                pos_v[2 * tt, sl] = (c * x - sn * y) + cx
                pos_v[2 * tt + 1, sl] = (sn * x + c * y) + cy
                val = w + sy
                yaw_v[tt, sl] = _wrap_angle_fast(val + PI)
            return 0

        lax.fori_loop(0, N_FUT, step, 0)
        cp_pos = pltpu.async_copy(pos_v, pos_hbm.at[s, wid], sem_pos)
        cp_yaw = pltpu.async_copy(yaw_v, yaw_hbm.at[s, wid], sem_yaw)
    cp_pos.wait()
    cp_yaw.wait()


def _sc_call(pred_r, idx_flat, trig_flat):
    mesh = plsc.VectorSubcoreMesh(core_axis_name="c", subcore_axis_name="s")
    f = pl.kernel(
        _sc_body,
        out_type=(
            jax.ShapeDtypeStruct((N_SC, N_JOINT, N_FUT * 2, N_AG), jnp.float32),
            jax.ShapeDtypeStruct((N_SC, N_JOINT, N_FUT, N_AG), jnp.float32),
        ),
        mesh=mesh,
        compiler_params=pltpu.CompilerParams(needs_layout_passes=False),
        scratch_types=[
            pltpu.VMEM((N_FUT * 3, N_AG), jnp.float32),
            pltpu.VMEM((N_FUT * 3, N_AG), jnp.float32),
            pltpu.VMEM((N_FUT * 2, N_AG), jnp.float32),
            pltpu.VMEM((N_FUT, N_AG), jnp.float32),
            pltpu.VMEM((N_SC * N_JOINT,), jnp.int32),
            pltpu.VMEM((N_SC * 128,), jnp.float32),
            pltpu.SemaphoreType.DMA,
            pltpu.SemaphoreType.DMA,
            pltpu.SemaphoreType.DMA,
            pltpu.SemaphoreType.DMA,
        ],
    )
    return f(pred_r, idx_flat, trig_flat)


def kernel(pred_pose, scenario_center, scenario_yaw, agent_pos_hist,
           no_sim_pos, no_sim_yaw, ag_role, collided, run_road_edge,
           valid_sim, valid_no_sim, object_id_sim, object_id_no_sim,
           scenario_id):
    coll8 = jnp.transpose(collided, (0, 3, 1, 2)).astype(jnp.int8)
    rre8 = jnp.transpose(run_road_edge, (0, 3, 1, 2)).astype(jnp.int8)
    role8 = jnp.swapaxes(ag_role, 1, 2).astype(jnp.int8)
    yaw3 = scenario_yaw.reshape(N_SC, 1, 1)
    cen3 = scenario_center.reshape(N_SC, 1, 2)
    nspos = no_sim_pos.reshape(N_SC, N_NOSIM, N_HIST * 3)
    nsyaw = no_sim_yaw.reshape(N_SC, N_NOSIM, N_HIST)

    idx3, trig3, posns, yawns = _prep_call(
        coll8, rre8, role8, yaw3, cen3, nspos, nsyaw)

    pred_r = jnp.transpose(pred_pose, (0, 3, 4, 1, 2)).reshape(
        N_SC, ROW3, N_K, N_AG)
    pos_flat, yaw_flat = _sc_call(
        pred_r, idx3.reshape(N_SC * N_JOINT), trig3.reshape(N_SC * 128))

    pos_sim = jnp.transpose(
        pos_flat.reshape(N_SC, N_JOINT, N_FUT, 2, N_AG), (0, 1, 4, 2, 3))
    yaw_sim = jnp.transpose(
        yaw_flat.reshape(N_SC, N_JOINT, N_FUT, 1, N_AG), (0, 1, 4, 2, 3))
    pos_ns = posns.reshape(N_SC, N_NOSIM, N_HIST, 2)
    yaw_ns = yawns.reshape(N_SC, N_NOSIM, N_HIST, 1)
    z_sim = agent_pos_hist[..., 2:3]
    z_ns = no_sim_pos[..., 2:3]
    return (scenario_id, valid_sim, pos_sim, z_sim, yaw_sim, valid_no_sim,
            object_id_sim, pos_ns, z_ns, yaw_ns, object_id_no_sim)


# speedup vs baseline: 5.9348x; 1.5380x over previous
import jax
import jax.numpy as jnp
import numpy as np
from jax import lax
from jax.experimental import pallas as pl
from jax.experimental.pallas import tpu as pltpu
from jax.experimental.pallas import tpu_sc as plsc

N_SC = 8
N_K = 64
N_AG = 128
N_STEP = 91
STEP_FUT = 11
N_JOINT = 32
N_HIST = 11
N_NOSIM = 64

N_FUT = N_STEP - STEP_FUT
ROW3 = N_STEP * 3

PI = np.float32(np.pi)
TWO_PI = np.float32(2.0 * np.pi)
INV_2PI = np.float32(1.0 / (2.0 * np.pi))

_NC = 2
_NS = 16


def _wrap_angle(v):
    m = lax.rem(v, TWO_PI)
    m = jnp.where((m != 0.0) & (m < 0.0), m + TWO_PI, m)
    return m - PI


def _wrap_angle_fast(v):
    q = (v * INV_2PI).astype(jnp.int32).astype(jnp.float32)
    m = v - q * TWO_PI
    m = jnp.where(m < 0.0, m + TWO_PI, m)
    return m - PI


def _prep_body(coll_ref, rre_ref, role_ref, yaw_ref, cen_ref, nspos_ref,
               nsyaw_ref, idx_ref, trig_ref, posns_ref, yawns_ref):
    col_any = jnp.max(coll_ref[0, STEP_FUT:, :, :].astype(jnp.float32), axis=0)
    rre_any = jnp.max(rre_ref[0, STEP_FUT:, :, :].astype(jnp.float32), axis=0)
    role = jnp.max(role_ref[0].astype(jnp.float32), axis=0, keepdims=True)
    cnt = (col_any + rre_any) * jnp.broadcast_to(role, (N_K, N_AG))
    ones_col = jnp.ones((N_AG, 1), jnp.float32)
    viol_col = lax.dot_general(cnt, ones_col, (((1,), (0,)), ((), ())),
                               precision=lax.Precision.HIGHEST)
    ones_row = jnp.ones((1, N_AG), jnp.float32)
    viol_row = lax.dot_general(ones_row, cnt, (((1,), (1,)), ((), ())),
                               precision=lax.Precision.HIGHEST)

    vi = jnp.broadcast_to(viol_col, (N_K, N_K))
    vj = jnp.broadcast_to(viol_row, (N_K, N_K))
    ii = lax.broadcasted_iota(jnp.int32, (N_K, N_K), 0)
    jj = lax.broadcasted_iota(jnp.int32, (N_K, N_K), 1)
    prec = (vj < vi) | ((vj == vi) & (jj < ii))
    rank_col = jnp.sum(prec.astype(jnp.int32), axis=1, keepdims=True)

    jj2 = lax.broadcasted_iota(jnp.int32, (N_K, N_JOINT), 1)
    ii2 = lax.broadcasted_iota(jnp.int32, (N_K, N_JOINT), 0)
    onehot = jnp.broadcast_to(rank_col, (N_K, N_JOINT)) == jj2
    idx_row = jnp.sum(jnp.where(onehot, ii2, 0), axis=0, keepdims=True)
    idx_ref[...] = idx_row.reshape(1, 1, N_JOINT)

    yaw = yaw_ref[0]
    c = jnp.cos(yaw)
    s = jnp.sin(yaw)
    cx = cen_ref[0, :, 0:1]
    cy = cen_ref[0, :, 1:2]
    rr8 = lax.broadcasted_iota(jnp.int32, (8, 16), 0)
    trig = jnp.where(rr8 == 0, jnp.broadcast_to(c, (8, 16)),
           jnp.where(rr8 == 1, jnp.broadcast_to(s, (8, 16)),
           jnp.where(rr8 == 2, jnp.broadcast_to(cx, (8, 16)),
           jnp.where(rr8 == 3, jnp.broadcast_to(cy, (8, 16)),
                     jnp.broadcast_to(yaw, (8, 16))))))
    trig_ref[...] = trig.reshape(1, 8, 16)

    NL = N_HIST * 3
    NQ = N_HIST * 2
    r0 = lax.broadcasted_iota(jnp.int32, (NL, NQ), 0)
    q0 = lax.broadcasted_iota(jnp.int32, (NL, NQ), 1)
    tmatch = (r0 // 3) == (q0 // 2)
    r3 = r0 % 3
    q2 = q0 % 2
    cb = jnp.broadcast_to(c, (NL, NQ))
    sb = jnp.broadcast_to(s, (NL, NQ))
    zero = jnp.zeros((NL, NQ), jnp.float32)
    m = jnp.where(tmatch & (r3 == 0) & (q2 == 0), cb, zero)
    m = m + jnp.where(tmatch & (r3 == 0) & (q2 == 1), sb, zero)
    m = m + jnp.where(tmatch & (r3 == 1) & (q2 == 0), -sb, zero)
    m = m + jnp.where(tmatch & (r3 == 1) & (q2 == 1), cb, zero)
    ns = nspos_ref[0]
    pos = lax.dot_general(ns, m, (((1,), (0,)), ((), ())),
                          precision=lax.Precision.HIGHEST)
    q2r = lax.broadcasted_iota(jnp.int32, (1, NQ), 1) % 2
    bias = jnp.where(q2r == 0, jnp.broadcast_to(cx, (1, NQ)),
                     jnp.broadcast_to(cy, (1, NQ)))
    posns_ref[...] = (pos + bias).reshape(1, N_NOSIM, NQ)

    yv = nsyaw_ref[0] + jnp.broadcast_to(yaw, (N_NOSIM, N_HIST))
    yawns_ref[...] = _wrap_angle(yv + PI).reshape(1, N_NOSIM, N_HIST)


def _prep_call(coll, rre, role_t, yaw3, cen3, nspos, nsyaw):
    return pl.pallas_call(
        _prep_body,
        grid=(N_SC,),
        in_specs=[
            pl.BlockSpec((1, N_STEP, N_K, N_AG), lambda s: (s, 0, 0, 0)),
            pl.BlockSpec((1, N_STEP, N_K, N_AG), lambda s: (s, 0, 0, 0)),
            pl.BlockSpec((1, 3, N_AG), lambda s: (s, 0, 0)),
            pl.BlockSpec((1, 1, 1), lambda s: (s, 0, 0)),
            pl.BlockSpec((1, 1, 2), lambda s: (s, 0, 0)),
            pl.BlockSpec((1, N_NOSIM, N_HIST * 3), lambda s: (s, 0, 0)),
            pl.BlockSpec((1, N_NOSIM, N_HIST), lambda s: (s, 0, 0)),
        ],
        out_specs=[
            pl.BlockSpec((1, 1, N_JOINT), lambda s: (s, 0, 0)),
            pl.BlockSpec((1, 8, 16), lambda s: (s, 0, 0)),
            pl.BlockSpec((1, N_NOSIM, N_HIST * 2), lambda s: (s, 0, 0)),
            pl.BlockSpec((1, N_NOSIM, N_HIST), lambda s: (s, 0, 0)),
        ],
        out_shape=[
            jax.ShapeDtypeStruct((N_SC, 1, N_JOINT), jnp.int32),
            jax.ShapeDtypeStruct((N_SC, 8, 16), jnp.float32),
            jax.ShapeDtypeStruct((N_SC, N_NOSIM, N_HIST * 2), jnp.float32),
            jax.ShapeDtypeStruct((N_SC, N_NOSIM, N_HIST), jnp.float32),
        ],
    )(coll, rre, role_t, yaw3, cen3, nspos, nsyaw)


def _sc_body(pred_hbm, idx_hbm, trig_hbm, pos_hbm, yaw_hbm,
             in_v0, in_v1, pos_v, yaw_v, idx_v, trig_v,
             sem_in0, sem_in1, sem_pos, sem_yaw):
    wid = lax.axis_index("s") * _NC + lax.axis_index("c")
    pltpu.sync_copy(idx_hbm, idx_v)
    pltpu.sync_copy(trig_hbm, trig_v)
    bufs = (in_v0, in_v1)
    sems = (sem_in0, sem_in1)

    def start_in(s):
        t = s * N_JOINT + wid
        k = plsc.load_gather(idx_v, [jnp.full((16,), 0, jnp.int32) + t])[0]
        return pltpu.async_copy(
            pred_hbm.at[s, pl.ds(STEP_FUT * 3, N_FUT * 3), k],
            bufs[s % 2], sems[s % 2])

    cp_in = start_in(0)
    cp_pos = cp_yaw = None
    for s in range(N_SC):
        buf = bufs[s % 2]
        cp_in.wait()
        if s + 1 < N_SC:
            cp_in = start_in(s + 1)
        base = s * 128
        c = trig_v[pl.ds(base, 16)]
        sn = trig_v[pl.ds(base + 16, 16)]
        cx = trig_v[pl.ds(base + 32, 16)]
        cy = trig_v[pl.ds(base + 48, 16)]
        sy = trig_v[pl.ds(base + 64, 16)]
        if cp_pos is not None:
            cp_pos.wait()
            cp_yaw.wait()

        @plsc.parallel_loop(0, N_FUT, unroll=4)
        def _step(tt, buf=buf, c=c, sn=sn, cx=cx, cy=cy, sy=sy):
            r = tt * 3
            for g in range(N_AG // 16):
                sl = pl.ds(16 * g, 16)
                x = buf[r, sl]
                y = buf[r + 1, sl]
                w = buf[r + 2, sl]
                pos_v[2 * tt, sl] = (c * x - sn * y) + cx
                pos_v[2 * tt + 1, sl] = (sn * x + c * y) + cy
                val = w + sy
                yaw_v[tt, sl] = _wrap_angle_fast(val + PI)

        cp_pos = pltpu.async_copy(pos_v, pos_hbm.at[s, wid], sem_pos)
        cp_yaw = pltpu.async_copy(yaw_v, yaw_hbm.at[s, wid], sem_yaw)
    cp_pos.wait()
    cp_yaw.wait()


def _sc_call(pred_r, idx_flat, trig_flat):
    mesh = plsc.VectorSubcoreMesh(core_axis_name="c", subcore_axis_name="s")
    f = pl.kernel(
        _sc_body,
        out_type=(
            jax.ShapeDtypeStruct((N_SC, N_JOINT, N_FUT * 2, N_AG), jnp.float32),
            jax.ShapeDtypeStruct((N_SC, N_JOINT, N_FUT, N_AG), jnp.float32),
        ),
        mesh=mesh,
        compiler_params=pltpu.CompilerParams(needs_layout_passes=False),
        scratch_types=[
            pltpu.VMEM((N_FUT * 3, N_AG), jnp.float32),
            pltpu.VMEM((N_FUT * 3, N_AG), jnp.float32),
            pltpu.VMEM((N_FUT * 2, N_AG), jnp.float32),
            pltpu.VMEM((N_FUT, N_AG), jnp.float32),
            pltpu.VMEM((N_SC * N_JOINT,), jnp.int32),
            pltpu.VMEM((N_SC * 128,), jnp.float32),
            pltpu.SemaphoreType.DMA,
            pltpu.SemaphoreType.DMA,
            pltpu.SemaphoreType.DMA,
            pltpu.SemaphoreType.DMA,
        ],
    )
    return f(pred_r, idx_flat, trig_flat)


def kernel(pred_pose, scenario_center, scenario_yaw, agent_pos_hist,
           no_sim_pos, no_sim_yaw, ag_role, collided, run_road_edge,
           valid_sim, valid_no_sim, object_id_sim, object_id_no_sim,
           scenario_id):
    coll8 = jnp.transpose(collided, (0, 3, 1, 2)).astype(jnp.int8)
    rre8 = jnp.transpose(run_road_edge, (0, 3, 1, 2)).astype(jnp.int8)
    role8 = jnp.swapaxes(ag_role, 1, 2).astype(jnp.int8)
    yaw3 = scenario_yaw.reshape(N_SC, 1, 1)
    cen3 = scenario_center.reshape(N_SC, 1, 2)
    nspos = no_sim_pos.reshape(N_SC, N_NOSIM, N_HIST * 3)
    nsyaw = no_sim_yaw.reshape(N_SC, N_NOSIM, N_HIST)

    idx3, trig3, posns, yawns = _prep_call(
        coll8, rre8, role8, yaw3, cen3, nspos, nsyaw)

    pred_r = jnp.transpose(pred_pose, (0, 3, 4, 1, 2)).reshape(
        N_SC, ROW3, N_K, N_AG)
    pos_flat, yaw_flat = _sc_call(
        pred_r, idx3.reshape(N_SC * N_JOINT), trig3.reshape(N_SC * 128))

    pos_sim = jnp.transpose(
        pos_flat.reshape(N_SC, N_JOINT, N_FUT, 2, N_AG), (0, 1, 4, 2, 3))
    yaw_sim = jnp.transpose(
        yaw_flat.reshape(N_SC, N_JOINT, N_FUT, 1, N_AG), (0, 1, 4, 2, 3))
    pos_ns = posns.reshape(N_SC, N_NOSIM, N_HIST, 2)
    yaw_ns = yawns.reshape(N_SC, N_NOSIM, N_HIST, 1)
    z_sim = agent_pos_hist[..., 2:3]
    z_ns = no_sim_pos[..., 2:3]
    return (scenario_id, valid_sim, pos_sim, z_sim, yaw_sim, valid_no_sim,
            object_id_sim, pos_ns, z_ns, yaw_ns, object_id_no_sim)


# speedup vs baseline: 6.5605x; 1.1054x over previous
import jax
import jax.numpy as jnp
import numpy as np
from jax import lax
from jax.experimental import pallas as pl
from jax.experimental.pallas import tpu as pltpu
from jax.experimental.pallas import tpu_sc as plsc

N_SC = 8
N_K = 64
N_AG = 128
N_STEP = 91
STEP_FUT = 11
N_JOINT = 32
N_HIST = 11
N_NOSIM = 64

N_FUT = N_STEP - STEP_FUT
ROW3 = N_STEP * 3

PI = np.float32(np.pi)
TWO_PI = np.float32(2.0 * np.pi)
INV_2PI = np.float32(1.0 / (2.0 * np.pi))

_NC = 2
_NS = 16


def _wrap_angle(v):
    m = lax.rem(v, TWO_PI)
    m = jnp.where((m != 0.0) & (m < 0.0), m + TWO_PI, m)
    return m - PI


def _wrap_angle_fast(v):
    q = (v * INV_2PI).astype(jnp.int32).astype(jnp.float32)
    m = v - q * TWO_PI
    m = jnp.where(m < 0.0, m + TWO_PI, m)
    return m - PI


def _prep_body(coll_ref, rre_ref, role_ref, yaw_ref, cen_ref, nspos_ref,
               nsyaw_ref, idx_ref, trig_ref, posns_ref, yawns_ref):
    def _or_tree(x):
        n = x.shape[0]
        while n % 2 == 0:
            h = n // 2
            x = x[0:h] | x[h:n]
            n = h
        r = x[0]
        for i in range(1, n):
            r = r | x[i]
        return r

    col_any = _or_tree(coll_ref[0, STEP_FUT:, :, :]).astype(jnp.float32)
    rre_any = _or_tree(rre_ref[0, STEP_FUT:, :, :]).astype(jnp.float32)
    role = (role_ref[0, 0] | role_ref[0, 1] | role_ref[0, 2]).astype(
        jnp.float32)[None, :]
    cnt = (col_any + rre_any) * jnp.broadcast_to(role, (N_K, N_AG))
    ones_col = jnp.ones((N_AG, 1), jnp.float32)
    viol_col = lax.dot_general(cnt, ones_col, (((1,), (0,)), ((), ())),
                               precision=lax.Precision.HIGHEST)
    ones_row = jnp.ones((1, N_AG), jnp.float32)
    viol_row = lax.dot_general(ones_row, cnt, (((1,), (1,)), ((), ())),
                               precision=lax.Precision.HIGHEST)

    vi = jnp.broadcast_to(viol_col, (N_K, N_K))
    vj = jnp.broadcast_to(viol_row, (N_K, N_K))
    ii = lax.broadcasted_iota(jnp.int32, (N_K, N_K), 0)
    jj = lax.broadcasted_iota(jnp.int32, (N_K, N_K), 1)
    prec = (vj < vi) | ((vj == vi) & (jj < ii))
    rank_col = jnp.sum(prec.astype(jnp.int32), axis=1, keepdims=True)

    jj2 = lax.broadcasted_iota(jnp.int32, (N_K, N_JOINT), 1)
    ii2 = lax.broadcasted_iota(jnp.int32, (N_K, N_JOINT), 0)
    onehot = jnp.broadcast_to(rank_col, (N_K, N_JOINT)) == jj2
    idx_row = jnp.sum(jnp.where(onehot, ii2, 0), axis=0, keepdims=True)
    idx_ref[...] = idx_row.reshape(1, 1, N_JOINT)

    yaw = yaw_ref[0]
    c = jnp.cos(yaw)
    s = jnp.sin(yaw)
    cx = cen_ref[0, :, 0:1]
    cy = cen_ref[0, :, 1:2]
    rr8 = lax.broadcasted_iota(jnp.int32, (8, 16), 0)
    trig = jnp.where(rr8 == 0, jnp.broadcast_to(c, (8, 16)),
           jnp.where(rr8 == 1, jnp.broadcast_to(s, (8, 16)),
           jnp.where(rr8 == 2, jnp.broadcast_to(cx, (8, 16)),
           jnp.where(rr8 == 3, jnp.broadcast_to(cy, (8, 16)),
                     jnp.broadcast_to(yaw, (8, 16))))))
    trig_ref[...] = trig.reshape(1, 8, 16)

    NL = N_HIST * 3
    NQ = N_HIST * 2
    r0 = lax.broadcasted_iota(jnp.int32, (NL, NQ), 0)
    q0 = lax.broadcasted_iota(jnp.int32, (NL, NQ), 1)
    tmatch = (r0 // 3) == (q0 // 2)
    r3 = r0 % 3
    q2 = q0 % 2
    cb = jnp.broadcast_to(c, (NL, NQ))
    sb = jnp.broadcast_to(s, (NL, NQ))
    zero = jnp.zeros((NL, NQ), jnp.float32)
    m = jnp.where(tmatch & (r3 == 0) & (q2 == 0), cb, zero)
    m = m + jnp.where(tmatch & (r3 == 0) & (q2 == 1), sb, zero)
    m = m + jnp.where(tmatch & (r3 == 1) & (q2 == 0), -sb, zero)
    m = m + jnp.where(tmatch & (r3 == 1) & (q2 == 1), cb, zero)
    ns = nspos_ref[0]
    pos = lax.dot_general(ns, m, (((1,), (0,)), ((), ())),
                          precision=lax.Precision.HIGHEST)
    q2r = lax.broadcasted_iota(jnp.int32, (1, NQ), 1) % 2
    bias = jnp.where(q2r == 0, jnp.broadcast_to(cx, (1, NQ)),
                     jnp.broadcast_to(cy, (1, NQ)))
    posns_ref[...] = (pos + bias).reshape(1, N_NOSIM, NQ)

    yv = nsyaw_ref[0] + jnp.broadcast_to(yaw, (N_NOSIM, N_HIST))
    yawns_ref[...] = _wrap_angle(yv + PI).reshape(1, N_NOSIM, N_HIST)


def _prep_call(coll, rre, role_t, yaw3, cen3, nspos, nsyaw):
    return pl.pallas_call(
        _prep_body,
        grid=(N_SC,),
        in_specs=[
            pl.BlockSpec((1, N_STEP, N_K, N_AG), lambda s: (s, 0, 0, 0)),
            pl.BlockSpec((1, N_STEP, N_K, N_AG), lambda s: (s, 0, 0, 0)),
            pl.BlockSpec((1, 3, N_AG), lambda s: (s, 0, 0)),
            pl.BlockSpec((1, 1, 1), lambda s: (s, 0, 0)),
            pl.BlockSpec((1, 1, 2), lambda s: (s, 0, 0)),
            pl.BlockSpec((1, N_NOSIM, N_HIST * 3), lambda s: (s, 0, 0)),
            pl.BlockSpec((1, N_NOSIM, N_HIST), lambda s: (s, 0, 0)),
        ],
        out_specs=[
            pl.BlockSpec((1, 1, N_JOINT), lambda s: (s, 0, 0)),
            pl.BlockSpec((1, 8, 16), lambda s: (s, 0, 0)),
            pl.BlockSpec((1, N_NOSIM, N_HIST * 2), lambda s: (s, 0, 0)),
            pl.BlockSpec((1, N_NOSIM, N_HIST), lambda s: (s, 0, 0)),
        ],
        out_shape=[
            jax.ShapeDtypeStruct((N_SC, 1, N_JOINT), jnp.int32),
            jax.ShapeDtypeStruct((N_SC, 8, 16), jnp.float32),
            jax.ShapeDtypeStruct((N_SC, N_NOSIM, N_HIST * 2), jnp.float32),
            jax.ShapeDtypeStruct((N_SC, N_NOSIM, N_HIST), jnp.float32),
        ],
    )(coll, rre, role_t, yaw3, cen3, nspos, nsyaw)


def _sc_body(pred_hbm, idx_hbm, trig_hbm, pos_hbm, yaw_hbm,
             in_v0, in_v1, pos_v0, pos_v1, yaw_v0, yaw_v1, idx_v, trig_v,
             sem_in0, sem_in1, sem_pos0, sem_pos1, sem_yaw0, sem_yaw1):
    wid = lax.axis_index("s") * _NC + lax.axis_index("c")
    pltpu.sync_copy(idx_hbm, idx_v)
    pltpu.sync_copy(trig_hbm, trig_v)
    bufs = (in_v0, in_v1)
    sems = (sem_in0, sem_in1)
    pos_bufs = (pos_v0, pos_v1)
    yaw_bufs = (yaw_v0, yaw_v1)
    sems_pos = (sem_pos0, sem_pos1)
    sems_yaw = (sem_yaw0, sem_yaw1)

    def start_in(s):
        t = s * N_JOINT + wid
        k = plsc.load_gather(idx_v, [jnp.full((16,), 0, jnp.int32) + t])[0]
        return pltpu.async_copy(
            pred_hbm.at[s, pl.ds(STEP_FUT * 3, N_FUT * 3), k],
            bufs[s % 2], sems[s % 2])

    cp_in = start_in(0)
    cp_out = [None, None]
    for s in range(N_SC):
        buf = bufs[s % 2]
        pos_v = pos_bufs[s % 2]
        yaw_v = yaw_bufs[s % 2]
        cp_in.wait()
        if s + 1 < N_SC:
            cp_in = start_in(s + 1)
        base = s * 128
        c = trig_v[pl.ds(base, 16)]
        sn = trig_v[pl.ds(base + 16, 16)]
        cx = trig_v[pl.ds(base + 32, 16)]
        cy = trig_v[pl.ds(base + 48, 16)]
        sy = trig_v[pl.ds(base + 64, 16)]
        if cp_out[s % 2] is not None:
            cp_out[s % 2][0].wait()
            cp_out[s % 2][1].wait()

        @plsc.parallel_loop(0, N_FUT, unroll=4)
        def _step(tt, buf=buf, pos_v=pos_v, yaw_v=yaw_v,
                  c=c, sn=sn, cx=cx, cy=cy, sy=sy):
            r = tt * 3
            for g in range(N_AG // 16):
                sl = pl.ds(16 * g, 16)
                x = buf[r, sl]
                y = buf[r + 1, sl]
                w = buf[r + 2, sl]
                pos_v[2 * tt, sl] = (c * x - sn * y) + cx
                pos_v[2 * tt + 1, sl] = (sn * x + c * y) + cy
                val = w + sy
                yaw_v[tt, sl] = _wrap_angle_fast(val + PI)

        cp_out[s % 2] = (
            pltpu.async_copy(pos_v, pos_hbm.at[s, wid], sems_pos[s % 2]),
            pltpu.async_copy(yaw_v, yaw_hbm.at[s, wid], sems_yaw[s % 2]))
    for pair in cp_out:
        if pair is not None:
            pair[0].wait()
            pair[1].wait()


def _sc_call(pred_r, idx_flat, trig_flat):
    mesh = plsc.VectorSubcoreMesh(core_axis_name="c", subcore_axis_name="s")
    f = pl.kernel(
        _sc_body,
        out_type=(
            jax.ShapeDtypeStruct((N_SC, N_JOINT, N_FUT * 2, N_AG), jnp.float32),
            jax.ShapeDtypeStruct((N_SC, N_JOINT, N_FUT, N_AG), jnp.float32),
        ),
        mesh=mesh,
        compiler_params=pltpu.CompilerParams(needs_layout_passes=False),
        scratch_types=[
            pltpu.VMEM((N_FUT * 3, N_AG), jnp.float32),
            pltpu.VMEM((N_FUT * 3, N_AG), jnp.float32),
            pltpu.VMEM((N_FUT * 2, N_AG), jnp.float32),
            pltpu.VMEM((N_FUT * 2, N_AG), jnp.float32),
            pltpu.VMEM((N_FUT, N_AG), jnp.float32),
            pltpu.VMEM((N_FUT, N_AG), jnp.float32),
            pltpu.VMEM((N_SC * N_JOINT,), jnp.int32),
            pltpu.VMEM((N_SC * 128,), jnp.float32),
            pltpu.SemaphoreType.DMA,
            pltpu.SemaphoreType.DMA,
            pltpu.SemaphoreType.DMA,
            pltpu.SemaphoreType.DMA,
            pltpu.SemaphoreType.DMA,
            pltpu.SemaphoreType.DMA,
        ],
    )
    return f(pred_r, idx_flat, trig_flat)


def kernel(pred_pose, scenario_center, scenario_yaw, agent_pos_hist,
           no_sim_pos, no_sim_yaw, ag_role, collided, run_road_edge,
           valid_sim, valid_no_sim, object_id_sim, object_id_no_sim,
           scenario_id):
    coll8 = jnp.transpose(collided, (0, 3, 1, 2)).astype(jnp.int8)
    rre8 = jnp.transpose(run_road_edge, (0, 3, 1, 2)).astype(jnp.int8)
    role8 = jnp.swapaxes(ag_role, 1, 2).astype(jnp.int8)
    yaw3 = scenario_yaw.reshape(N_SC, 1, 1)
    cen3 = scenario_center.reshape(N_SC, 1, 2)
    nspos = no_sim_pos.reshape(N_SC, N_NOSIM, N_HIST * 3)
    nsyaw = no_sim_yaw.reshape(N_SC, N_NOSIM, N_HIST)

    idx3, trig3, posns, yawns = _prep_call(
        coll8, rre8, role8, yaw3, cen3, nspos, nsyaw)

    pred_r = jnp.transpose(pred_pose, (0, 3, 4, 1, 2)).reshape(
        N_SC, ROW3, N_K, N_AG)
    pos_flat, yaw_flat = _sc_call(
        pred_r, idx3.reshape(N_SC * N_JOINT), trig3.reshape(N_SC * 128))

    pos_sim = jnp.transpose(
        pos_flat.reshape(N_SC, N_JOINT, N_FUT, 2, N_AG), (0, 1, 4, 2, 3))
    yaw_sim = jnp.transpose(
        yaw_flat.reshape(N_SC, N_JOINT, N_FUT, 1, N_AG), (0, 1, 4, 2, 3))
    pos_ns = posns.reshape(N_SC, N_NOSIM, N_HIST, 2)
    yaw_ns = yawns.reshape(N_SC, N_NOSIM, N_HIST, 1)
    z_sim = agent_pos_hist[..., 2:3]
    z_ns = no_sim_pos[..., 2:3]
    return (scenario_id, valid_sim, pos_sim, z_sim, yaw_sim, valid_no_sim,
            object_id_sim, pos_ns, z_ns, yaw_ns, object_id_no_sim)


# speedup vs baseline: 6.6223x; 1.0094x over previous
import jax
import jax.numpy as jnp
import numpy as np
from jax import lax
from jax.experimental import pallas as pl
from jax.experimental.pallas import tpu as pltpu
from jax.experimental.pallas import tpu_sc as plsc

N_SC = 8
N_K = 64
N_AG = 128
N_STEP = 91
STEP_FUT = 11
N_JOINT = 32
N_HIST = 11
N_NOSIM = 64

N_FUT = N_STEP - STEP_FUT
ROW3 = N_STEP * 3

PI = np.float32(np.pi)
TWO_PI = np.float32(2.0 * np.pi)
INV_2PI = np.float32(1.0 / (2.0 * np.pi))

_NC = 2
_NS = 16


def _wrap_angle(v):
    m = lax.rem(v, TWO_PI)
    m = jnp.where((m != 0.0) & (m < 0.0), m + TWO_PI, m)
    return m - PI


def _wrap_angle_fast(v):
    q = (v * INV_2PI).astype(jnp.int32).astype(jnp.float32)
    m = v - q * TWO_PI
    m = jnp.where(m < 0.0, m + TWO_PI, m)
    return m - PI


def _prep_body(coll_ref, rre_ref, role_ref, yaw_ref, cen_ref, nspos_ref,
               nsyaw_ref, idx_ref, trig_ref, posns_ref, yawns_ref):
    def _or_tree(x):
        n = x.shape[0]
        while n % 2 == 0:
            h = n // 2
            x = x[0:h] | x[h:n]
            n = h
        r = x[0]
        for i in range(1, n):
            r = r | x[i]
        return r

    col_any = _or_tree(coll_ref[0]).astype(jnp.float32)
    rre_any = _or_tree(rre_ref[0]).astype(jnp.float32)
    role = (role_ref[0, 0] | role_ref[0, 1] | role_ref[0, 2]).astype(
        jnp.float32)[None, :]
    cnt = (col_any + rre_any) * jnp.broadcast_to(role, (N_K, N_AG))
    ones_col = jnp.ones((N_AG, 1), jnp.float32)
    viol_col = lax.dot_general(cnt, ones_col, (((1,), (0,)), ((), ())),
                               precision=lax.Precision.HIGHEST)
    ones_row = jnp.ones((1, N_AG), jnp.float32)
    viol_row = lax.dot_general(ones_row, cnt, (((1,), (1,)), ((), ())),
                               precision=lax.Precision.HIGHEST)

    vi = jnp.broadcast_to(viol_col, (N_K, N_K))
    vj = jnp.broadcast_to(viol_row, (N_K, N_K))
    ii = lax.broadcasted_iota(jnp.int32, (N_K, N_K), 0)
    jj = lax.broadcasted_iota(jnp.int32, (N_K, N_K), 1)
    prec = (vj < vi) | ((vj == vi) & (jj < ii))
    rank_col = jnp.sum(prec.astype(jnp.int32), axis=1, keepdims=True)

    jj2 = lax.broadcasted_iota(jnp.int32, (N_K, N_JOINT), 1)
    ii2 = lax.broadcasted_iota(jnp.int32, (N_K, N_JOINT), 0)
    onehot = jnp.broadcast_to(rank_col, (N_K, N_JOINT)) == jj2
    idx_row = jnp.sum(jnp.where(onehot, ii2, 0), axis=0, keepdims=True)
    idx_ref[...] = idx_row.reshape(1, 1, N_JOINT)

    yaw = yaw_ref[0]
    c = jnp.cos(yaw)
    s = jnp.sin(yaw)
    cx = cen_ref[0, :, 0:1]
    cy = cen_ref[0, :, 1:2]
    rr8 = lax.broadcasted_iota(jnp.int32, (8, 16), 0)
    trig = jnp.where(rr8 == 0, jnp.broadcast_to(c, (8, 16)),
           jnp.where(rr8 == 1, jnp.broadcast_to(s, (8, 16)),
           jnp.where(rr8 == 2, jnp.broadcast_to(cx, (8, 16)),
           jnp.where(rr8 == 3, jnp.broadcast_to(cy, (8, 16)),
                     jnp.broadcast_to(yaw, (8, 16))))))
    trig_ref[...] = trig.reshape(1, 8, 16)

    NL = N_HIST * 3
    NQ = N_HIST * 2
    r0 = lax.broadcasted_iota(jnp.int32, (NL, NQ), 0)
    q0 = lax.broadcasted_iota(jnp.int32, (NL, NQ), 1)
    tmatch = (r0 // 3) == (q0 // 2)
    r3 = r0 % 3
    q2 = q0 % 2
    cb = jnp.broadcast_to(c, (NL, NQ))
    sb = jnp.broadcast_to(s, (NL, NQ))
    zero = jnp.zeros((NL, NQ), jnp.float32)
    m = jnp.where(tmatch & (r3 == 0) & (q2 == 0), cb, zero)
    m = m + jnp.where(tmatch & (r3 == 0) & (q2 == 1), sb, zero)
    m = m + jnp.where(tmatch & (r3 == 1) & (q2 == 0), -sb, zero)
    m = m + jnp.where(tmatch & (r3 == 1) & (q2 == 1), cb, zero)
    ns = nspos_ref[0]
    pos = lax.dot_general(ns, m, (((1,), (0,)), ((), ())),
                          precision=lax.Precision.HIGHEST)
    q2r = lax.broadcasted_iota(jnp.int32, (1, NQ), 1) % 2
    bias = jnp.where(q2r == 0, jnp.broadcast_to(cx, (1, NQ)),
                     jnp.broadcast_to(cy, (1, NQ)))
    posns_ref[...] = (pos + bias).reshape(1, N_NOSIM, NQ)

    yv = nsyaw_ref[0] + jnp.broadcast_to(yaw, (N_NOSIM, N_HIST))
    yawns_ref[...] = _wrap_angle(yv + PI).reshape(1, N_NOSIM, N_HIST)


def _prep_call(coll, rre, role_t, yaw3, cen3, nspos, nsyaw):
    return pl.pallas_call(
        _prep_body,
        grid=(N_SC,),
        in_specs=[
            pl.BlockSpec((1, N_FUT, N_K, N_AG), lambda s: (s, 0, 0, 0)),
            pl.BlockSpec((1, N_FUT, N_K, N_AG), lambda s: (s, 0, 0, 0)),
            pl.BlockSpec((1, 3, N_AG), lambda s: (s, 0, 0)),
            pl.BlockSpec((1, 1, 1), lambda s: (s, 0, 0)),
            pl.BlockSpec((1, 1, 2), lambda s: (s, 0, 0)),
            pl.BlockSpec((1, N_NOSIM, N_HIST * 3), lambda s: (s, 0, 0)),
            pl.BlockSpec((1, N_NOSIM, N_HIST), lambda s: (s, 0, 0)),
        ],
        out_specs=[
            pl.BlockSpec((1, 1, N_JOINT), lambda s: (s, 0, 0)),
            pl.BlockSpec((1, 8, 16), lambda s: (s, 0, 0)),
            pl.BlockSpec((1, N_NOSIM, N_HIST * 2), lambda s: (s, 0, 0)),
            pl.BlockSpec((1, N_NOSIM, N_HIST), lambda s: (s, 0, 0)),
        ],
        out_shape=[
            jax.ShapeDtypeStruct((N_SC, 1, N_JOINT), jnp.int32),
            jax.ShapeDtypeStruct((N_SC, 8, 16), jnp.float32),
            jax.ShapeDtypeStruct((N_SC, N_NOSIM, N_HIST * 2), jnp.float32),
            jax.ShapeDtypeStruct((N_SC, N_NOSIM, N_HIST), jnp.float32),
        ],
    )(coll, rre, role_t, yaw3, cen3, nspos, nsyaw)


def _sc_body(pred_hbm, idx_hbm, trig_hbm, pos_hbm, yaw_hbm,
             in_v0, in_v1, pos_v0, pos_v1, yaw_v0, yaw_v1, idx_v, trig_v,
             sem_in0, sem_in1, sem_pos0, sem_pos1, sem_yaw0, sem_yaw1):
    wid = lax.axis_index("s") * _NC + lax.axis_index("c")
    pltpu.sync_copy(idx_hbm, idx_v)
    pltpu.sync_copy(trig_hbm, trig_v)
    bufs = (in_v0, in_v1)
    sems = (sem_in0, sem_in1)
    pos_bufs = (pos_v0, pos_v1)
    yaw_bufs = (yaw_v0, yaw_v1)
    sems_pos = (sem_pos0, sem_pos1)
    sems_yaw = (sem_yaw0, sem_yaw1)

    def start_in(s):
        t = s * N_JOINT + wid
        k = plsc.load_gather(idx_v, [jnp.full((16,), 0, jnp.int32) + t])[0]
        return pltpu.async_copy(
            pred_hbm.at[s, pl.ds(STEP_FUT * 3, N_FUT * 3), k],
            bufs[s % 2], sems[s % 2])

    cp_in = start_in(0)
    cp_out = [None, None]
    for s in range(N_SC):
        buf = bufs[s % 2]
        pos_v = pos_bufs[s % 2]
        yaw_v = yaw_bufs[s % 2]
        cp_in.wait()
        if s + 1 < N_SC:
            cp_in = start_in(s + 1)
        base = s * 128
        c = trig_v[pl.ds(base, 16)]
        sn = trig_v[pl.ds(base + 16, 16)]
        cx = trig_v[pl.ds(base + 32, 16)]
        cy = trig_v[pl.ds(base + 48, 16)]
        sy = trig_v[pl.ds(base + 64, 16)]
        if cp_out[s % 2] is not None:
            cp_out[s % 2][0].wait()
            cp_out[s % 2][1].wait()

        @plsc.parallel_loop(0, N_FUT, unroll=8)
        def _step(tt, buf=buf, pos_v=pos_v, yaw_v=yaw_v,
                  c=c, sn=sn, cx=cx, cy=cy, sy=sy):
            r = tt * 3
            for g in range(N_AG // 16):
                sl = pl.ds(16 * g, 16)
                x = buf[r, sl]
                y = buf[r + 1, sl]
                w = buf[r + 2, sl]
                pos_v[2 * tt, sl] = (c * x - sn * y) + cx
                pos_v[2 * tt + 1, sl] = (sn * x + c * y) + cy
                val = w + sy
                yaw_v[tt, sl] = _wrap_angle_fast(val + PI)

        cp_out[s % 2] = (
            pltpu.async_copy(pos_v, pos_hbm.at[s, wid], sems_pos[s % 2]),
            pltpu.async_copy(yaw_v, yaw_hbm.at[s, wid], sems_yaw[s % 2]))
    for pair in cp_out:
        if pair is not None:
            pair[0].wait()
            pair[1].wait()


def _sc_call(pred_r, idx_flat, trig_flat):
    mesh = plsc.VectorSubcoreMesh(core_axis_name="c", subcore_axis_name="s")
    f = pl.kernel(
        _sc_body,
        out_type=(
            jax.ShapeDtypeStruct((N_SC, N_JOINT, N_FUT * 2, N_AG), jnp.float32),
            jax.ShapeDtypeStruct((N_SC, N_JOINT, N_FUT, N_AG), jnp.float32),
        ),
        mesh=mesh,
        compiler_params=pltpu.CompilerParams(needs_layout_passes=False),
        scratch_types=[
            pltpu.VMEM((N_FUT * 3, N_AG), jnp.float32),
            pltpu.VMEM((N_FUT * 3, N_AG), jnp.float32),
            pltpu.VMEM((N_FUT * 2, N_AG), jnp.float32),
            pltpu.VMEM((N_FUT * 2, N_AG), jnp.float32),
            pltpu.VMEM((N_FUT, N_AG), jnp.float32),
            pltpu.VMEM((N_FUT, N_AG), jnp.float32),
            pltpu.VMEM((N_SC * N_JOINT,), jnp.int32),
            pltpu.VMEM((N_SC * 128,), jnp.float32),
            pltpu.SemaphoreType.DMA,
            pltpu.SemaphoreType.DMA,
            pltpu.SemaphoreType.DMA,
            pltpu.SemaphoreType.DMA,
            pltpu.SemaphoreType.DMA,
            pltpu.SemaphoreType.DMA,
        ],
    )
    return f(pred_r, idx_flat, trig_flat)


def kernel(pred_pose, scenario_center, scenario_yaw, agent_pos_hist,
           no_sim_pos, no_sim_yaw, ag_role, collided, run_road_edge,
           valid_sim, valid_no_sim, object_id_sim, object_id_no_sim,
           scenario_id):
    coll8 = jnp.transpose(collided, (0, 3, 1, 2))[:, STEP_FUT:].astype(jnp.int8)
    rre8 = jnp.transpose(run_road_edge, (0, 3, 1, 2))[:, STEP_FUT:].astype(jnp.int8)
    role8 = jnp.swapaxes(ag_role, 1, 2).astype(jnp.int8)
    yaw3 = scenario_yaw.reshape(N_SC, 1, 1)
    cen3 = scenario_center.reshape(N_SC, 1, 2)
    nspos = no_sim_pos.reshape(N_SC, N_NOSIM, N_HIST * 3)
    nsyaw = no_sim_yaw.reshape(N_SC, N_NOSIM, N_HIST)

    idx3, trig3, posns, yawns = _prep_call(
        coll8, rre8, role8, yaw3, cen3, nspos, nsyaw)

    pred_r = jnp.transpose(pred_pose, (0, 3, 4, 1, 2)).reshape(
        N_SC, ROW3, N_K, N_AG)
    pos_flat, yaw_flat = _sc_call(
        pred_r, idx3.reshape(N_SC * N_JOINT), trig3.reshape(N_SC * 128))

    pos_sim = jnp.transpose(
        pos_flat.reshape(N_SC, N_JOINT, N_FUT, 2, N_AG), (0, 1, 4, 2, 3))
    yaw_sim = jnp.transpose(
        yaw_flat.reshape(N_SC, N_JOINT, N_FUT, 1, N_AG), (0, 1, 4, 2, 3))
    pos_ns = posns.reshape(N_SC, N_NOSIM, N_HIST, 2)
    yaw_ns = yawns.reshape(N_SC, N_NOSIM, N_HIST, 1)
    z_sim = agent_pos_hist[..., 2:3]
    z_ns = no_sim_pos[..., 2:3]
    return (scenario_id, valid_sim, pos_sim, z_sim, yaw_sim, valid_no_sim,
            object_id_sim, pos_ns, z_ns, yaw_ns, object_id_no_sim)
